# baseline (speedup 1.0000x reference)
import functools

import jax
import jax.numpy as jnp
from jax import lax
from jax.experimental import pallas as pl
from jax.experimental.pallas import tpu as pltpu
from jax.experimental.pallas import tpu_sc as plsc

N_NODES = 10000
N_EDGES = 320000
D_FEAT = 128
LANES = 16

NC = 2
NS = 16
NW = NC * NS

NPAD = 10240
BM = 512
BN = 2560
EDGES_PER_TILE = N_EDGES // NW


BC = 128
NJ = NPAD // BC
BLK = NPAD * BC


def _mm_body(m_ref, mt_ref, g_ref):
    res = lax.dot_general(
        m_ref[...], mt_ref[...], (((1,), (1,)), ((), ())),
        preferred_element_type=jnp.float32)
    bf = res.astype(jnp.bfloat16).reshape(NPAD // 2, 2, BC)
    lo = lax.bitcast_convert_type(bf[:, 0], jnp.uint16).astype(jnp.uint32)
    hi = lax.bitcast_convert_type(bf[:, 1], jnp.uint16).astype(jnp.uint32)
    w = lax.bitcast_convert_type(lo | (hi << 16), jnp.int32)
    g_ref[...] = w.reshape(BLK // 2)


def _gram(mp):
    return pl.pallas_call(
        _mm_body,
        grid=(NJ,),
        in_specs=[pl.BlockSpec((NPAD, D_FEAT), lambda j: (0, 0)),
                  pl.BlockSpec((BC, D_FEAT), lambda j: (j, 0))],
        out_specs=pl.BlockSpec((BLK // 2,), lambda j: (j,)),
        out_shape=jax.ShapeDtypeStruct((NPAD * NPAD // 2,), jnp.int32),
    )(mp, mp)


def _sc_body(src_hbm, dst_hbm, g_hbm, out_hbm, sidx, didx, widx, vals, outv,
             sem):
    cid = lax.axis_index("c")
    sid = lax.axis_index("s")
    wid = sid * NC + cid
    base = wid * EDGES_PER_TILE

    pltpu.sync_copy(src_hbm.at[pl.ds(base, EDGES_PER_TILE)], sidx)
    pltpu.sync_copy(dst_hbm.at[pl.ds(base, EDGES_PER_TILE)], didx)

    def flat_step(i, _):
        s = pl.ds(i * LANES, LANES)
        sv = sidx[s]
        dv = didx[s]
        widx[s] = (dv >> 7) * (BLK // 2) + ((sv >> 1) << 7) + (dv & (BC - 1))
        return 0

    lax.fori_loop(0, EDGES_PER_TILE // LANES, flat_step, 0)

    pltpu.async_copy(g_hbm.at[widx], vals, sem).wait()

    def unpack_step(i, _):
        s = pl.ds(i * LANES, LANES)
        w = plsc.bitcast(vals[s], jnp.bfloat16)
        lo, hi = plsc.unpack(w, format=plsc.PackFormat.INTERLEAVED)
        outv[s] = jnp.where(sidx[s] & 1 == 0, lo, hi)
        return 0

    lax.fori_loop(0, EDGES_PER_TILE // LANES, unpack_step, 0)
    pltpu.sync_copy(outv, out_hbm.at[pl.ds(base, EDGES_PER_TILE)])


def _sc_gather(src, dst, g):
    mesh = plsc.VectorSubcoreMesh(core_axis_name="c", subcore_axis_name="s")
    return pl.kernel(
        _sc_body,
        out_type=jax.ShapeDtypeStruct((N_EDGES,), jnp.float32),
        mesh=mesh,
        compiler_params=pltpu.CompilerParams(needs_layout_passes=False,
                                             use_tc_tiling_on_sc=False),
        scratch_types=[
            pltpu.VMEM((EDGES_PER_TILE,), jnp.int32),
            pltpu.VMEM((EDGES_PER_TILE,), jnp.int32),
            pltpu.VMEM((EDGES_PER_TILE,), jnp.int32),
            pltpu.VMEM((EDGES_PER_TILE,), jnp.int32),
            pltpu.VMEM((EDGES_PER_TILE,), jnp.float32),
            pltpu.SemaphoreType.DMA,
        ],
    )(src, dst, g)


@jax.jit
def _run(src, dst, model):
    mp = jnp.zeros((NPAD, D_FEAT), jnp.bfloat16)
    mp = lax.dynamic_update_slice(mp, model.astype(jnp.bfloat16), (0, 0))
    g = _gram(mp)
    return _sc_gather(src, dst, g)


def kernel(model, edge_index):
    ei = edge_index.astype(jnp.int32)
    return _run(ei[0], ei[1], model)


# speedup vs baseline: 3.4724x; 3.4724x over previous
import functools

import jax
import jax.numpy as jnp
from jax import lax
from jax.experimental import pallas as pl
from jax.experimental.pallas import tpu as pltpu
from jax.experimental.pallas import tpu_sc as plsc

N_NODES = 10000
N_EDGES = 320000
D_FEAT = 128
LANES = 16

NC = 2
NS = 16
NW = NC * NS

NPAD = 10240
BM = 512
BN = 2560
EDGES_PER_TILE = N_EDGES // NW


BC = 128
NJ = NPAD // BC
BLK = NPAD * BC


def _mm_body(m_ref, mt_ref, g_ref):
    res = lax.dot_general(
        m_ref[...], mt_ref[...], (((1,), (1,)), ((), ())),
        preferred_element_type=jnp.float32)
    bf = res.astype(jnp.bfloat16)
    lo = lax.bitcast_convert_type(bf[:NPAD // 2], jnp.uint16).astype(jnp.uint32)
    hi = lax.bitcast_convert_type(bf[NPAD // 2:], jnp.uint16).astype(jnp.uint32)
    w = lax.bitcast_convert_type(lo | (hi << 16), jnp.int32)
    g_ref[...] = w.reshape(BLK // 2)


def _gram(mp):
    return pl.pallas_call(
        _mm_body,
        grid=(NJ,),
        in_specs=[pl.BlockSpec((NPAD, D_FEAT), lambda j: (0, 0)),
                  pl.BlockSpec((BC, D_FEAT), lambda j: (j, 0))],
        out_specs=pl.BlockSpec((BLK // 2,), lambda j: (j,)),
        out_shape=jax.ShapeDtypeStruct((NPAD * NPAD // 2,), jnp.int32),
    )(mp, mp)


def _sc_body(src_hbm, dst_hbm, g_hbm, out_hbm, sidx, didx, widx, vals, outv,
             sem):
    cid = lax.axis_index("c")
    sid = lax.axis_index("s")
    wid = sid * NC + cid
    base = wid * EDGES_PER_TILE

    pltpu.sync_copy(src_hbm.at[pl.ds(base, EDGES_PER_TILE)], sidx)
    pltpu.sync_copy(dst_hbm.at[pl.ds(base, EDGES_PER_TILE)], didx)

    def flat_step(i, _):
        s = pl.ds(i * LANES, LANES)
        sv = sidx[s]
        dv = didx[s]
        sm = jnp.where(sv >= NPAD // 2, sv - NPAD // 2, sv)
        widx[s] = (dv >> 7) * (BLK // 2) + (sm << 7) + (dv & (BC - 1))
        return 0

    lax.fori_loop(0, EDGES_PER_TILE // LANES, flat_step, 0)

    pltpu.async_copy(g_hbm.at[widx], vals, sem).wait()

    def unpack_step(i, _):
        s = pl.ds(i * LANES, LANES)
        w = plsc.bitcast(vals[s], jnp.bfloat16)
        lo, hi = plsc.unpack(w, format=plsc.PackFormat.INTERLEAVED)
        outv[s] = jnp.where(sidx[s] < NPAD // 2, lo, hi)
        return 0

    lax.fori_loop(0, EDGES_PER_TILE // LANES, unpack_step, 0)
    pltpu.sync_copy(outv, out_hbm.at[pl.ds(base, EDGES_PER_TILE)])


def _sc_gather(src, dst, g):
    mesh = plsc.VectorSubcoreMesh(core_axis_name="c", subcore_axis_name="s")
    return pl.kernel(
        _sc_body,
        out_type=jax.ShapeDtypeStruct((N_EDGES,), jnp.float32),
        mesh=mesh,
        compiler_params=pltpu.CompilerParams(needs_layout_passes=False,
                                             use_tc_tiling_on_sc=False),
        scratch_types=[
            pltpu.VMEM((EDGES_PER_TILE,), jnp.int32),
            pltpu.VMEM((EDGES_PER_TILE,), jnp.int32),
            pltpu.VMEM((EDGES_PER_TILE,), jnp.int32),
            pltpu.VMEM((EDGES_PER_TILE,), jnp.int32),
            pltpu.VMEM((EDGES_PER_TILE,), jnp.float32),
            pltpu.SemaphoreType.DMA,
        ],
    )(src, dst, g)


@jax.jit
def _run(src, dst, model):
    mp = jnp.zeros((NPAD, D_FEAT), jnp.bfloat16)
    mp = lax.dynamic_update_slice(mp, model.astype(jnp.bfloat16), (0, 0))
    g = _gram(mp)
    return _sc_gather(src, dst, g)


def kernel(model, edge_index):
    ei = edge_index.astype(jnp.int32)
    return _run(ei[0], ei[1], model)


# speedup vs baseline: 3.4724x; 1.0000x over previous
import jax
import jax.numpy as jnp
from jax import lax
from jax.experimental import pallas as pl
from jax.experimental.pallas import tpu as pltpu
from jax.experimental.pallas import tpu_sc as plsc

N_NODES = 10000
N_EDGES = 320000
D_FEAT = 128
LANES = 16

NC = 2
NS = 16
NW = NC * NS

NPAD = 10240
EDGES_PER_TILE = N_EDGES // NW


BC = 128
NJ = NPAD // BC
BLK = NPAD * BC


def _mm_body(m_ref, mt_ref, g_ref):
    res = lax.dot_general(
        m_ref[...], mt_ref[...], (((1,), (1,)), ((), ())),
        preferred_element_type=jnp.float32)
    bf = res.astype(jnp.bfloat16)
    lo = lax.bitcast_convert_type(bf[:NPAD // 2], jnp.uint16).astype(jnp.uint32)
    hi = lax.bitcast_convert_type(bf[NPAD // 2:], jnp.uint16).astype(jnp.uint32)
    w = lax.bitcast_convert_type(lo | (hi << 16), jnp.int32)
    g_ref[...] = w.reshape(BLK // 2)


def _gram(mp):
    return pl.pallas_call(
        _mm_body,
        grid=(NJ,),
        in_specs=[pl.BlockSpec((NPAD, D_FEAT), lambda j: (0, 0)),
                  pl.BlockSpec((BC, D_FEAT), lambda j: (j, 0))],
        out_specs=pl.BlockSpec((BLK // 2,), lambda j: (j,)),
        out_shape=jax.ShapeDtypeStruct((NPAD * NPAD // 2,), jnp.int32),
    )(mp, mp)


def _sc_body(src_hbm, dst_hbm, g_hbm, out_hbm, sidx, didx, widx, vals, outv,
             sem):
    cid = lax.axis_index("c")
    sid = lax.axis_index("s")
    wid = sid * NC + cid
    base = wid * EDGES_PER_TILE

    pltpu.sync_copy(src_hbm.at[pl.ds(base, EDGES_PER_TILE)], sidx)
    pltpu.sync_copy(dst_hbm.at[pl.ds(base, EDGES_PER_TILE)], didx)

    def flat_step(i, _):
        s = pl.ds(i * LANES, LANES)
        sv = sidx[s]
        dv = didx[s]
        sm = jnp.where(sv >= NPAD // 2, sv - NPAD // 2, sv)
        widx[s] = (dv >> 7) * (BLK // 2) + (sm << 7) + (dv & (BC - 1))
        return 0

    lax.fori_loop(0, EDGES_PER_TILE // LANES, flat_step, 0)

    pltpu.async_copy(g_hbm.at[widx], vals, sem).wait()

    def unpack_step(i, _):
        s = pl.ds(i * LANES, LANES)
        w = plsc.bitcast(vals[s], jnp.bfloat16)
        lo, hi = plsc.unpack(w, format=plsc.PackFormat.INTERLEAVED)
        outv[s] = jnp.where(sidx[s] < NPAD // 2, lo, hi)
        return 0

    lax.fori_loop(0, EDGES_PER_TILE // LANES, unpack_step, 0)
    pltpu.sync_copy(outv, out_hbm.at[pl.ds(base, EDGES_PER_TILE)])


def _sc_gather(src, dst, g):
    mesh = plsc.VectorSubcoreMesh(core_axis_name="c", subcore_axis_name="s")
    return pl.kernel(
        _sc_body,
        out_type=jax.ShapeDtypeStruct((N_EDGES,), jnp.float32),
        mesh=mesh,
        compiler_params=pltpu.CompilerParams(needs_layout_passes=False,
                                             use_tc_tiling_on_sc=False),
        scratch_types=[
            pltpu.VMEM((EDGES_PER_TILE,), jnp.int32),
            pltpu.VMEM((EDGES_PER_TILE,), jnp.int32),
            pltpu.VMEM((EDGES_PER_TILE,), jnp.int32),
            pltpu.VMEM((EDGES_PER_TILE,), jnp.int32),
            pltpu.VMEM((EDGES_PER_TILE,), jnp.float32),
            pltpu.SemaphoreType.DMA,
        ],
    )(src, dst, g)


@jax.jit
def _run(src, dst, model):
    mp = jnp.zeros((NPAD, D_FEAT), jnp.bfloat16)
    mp = lax.dynamic_update_slice(mp, model.astype(jnp.bfloat16), (0, 0))
    g = _gram(mp)
    return _sc_gather(src, dst, g)


def kernel(model, edge_index):
    ei = edge_index.astype(jnp.int32)
    return _run(ei[0], ei[1], model)
